# baseline (speedup 1.0000x reference)
import functools

import jax
import jax.numpy as jnp
from jax import lax
from jax.experimental import pallas as pl
from jax.experimental.pallas import tpu as pltpu
from jax.experimental.pallas import tpu_sc as plsc

_NC = 2
_NS = 16
_NW = _NC * _NS
_IPD = 128
_DPC = 2


def _sc_gather_grp(table_grp, gidx3, sidx3, n_rows):
    rpw = n_rows // _NW
    n_dma = rpw // _IPD
    chunk_rows = _IPD * _DPC
    n_chunks = n_dma // _DPC
    mesh = plsc.VectorSubcoreMesh(core_axis_name="c", subcore_axis_name="s")

    @functools.partial(
        pl.kernel,
        out_type=jax.ShapeDtypeStruct((n_rows // 8, 128), jnp.float32),
        mesh=mesh,
        compiler_params=pltpu.CompilerParams(
            use_tc_tiling_on_sc=True, needs_layout_passes=False),
        scratch_types=[
            pltpu.VMEM((n_dma, _IPD), jnp.int32),
            pltpu.VMEM((n_dma, _IPD), jnp.int32),
            pltpu.VMEM((chunk_rows, 128), jnp.float32),
            pltpu.VMEM((chunk_rows // 8, 128), jnp.float32),
            pltpu.SemaphoreType.DMA,
        ],
    )
    def gather_k(table_hbm, gidx_hbm, sidx_hbm, out_hbm, gidx_v, sidx_v,
                 grp_v, out_v, gsem):
        wid = lax.axis_index("s") * _NC + lax.axis_index("c")
        row0 = wid * rpw
        pltpu.sync_copy(gidx_hbm.at[wid], gidx_v)
        pltpu.sync_copy(sidx_hbm.at[wid], sidx_v)
        lanes = lax.iota(jnp.int32, 16)

        def chunk_body(c, carry):
            copies = []
            for m in range(_DPC):
                cp = pltpu.async_copy(
                    table_hbm.at[gidx_v.at[c * _DPC + m]],
                    grp_v.at[pl.ds(m * _IPD, _IPD)],
                    gsem,
                )
                copies.append(cp)
            for cp in copies:
                cp.wait()

            def tile_body(m, carry2):
                for k in range(8):
                    s = (sidx_v[c * _DPC + m, pl.ds(k * 16, 16)] >> 3) & 7
                    rows = m * 128 + k * 16 + lanes
                    base = s * 16
                    for d in range(16):
                        vals = plsc.load_gather(grp_v, [rows, base + d])
                        fj = rows * 16 + d
                        plsc.store_scatter(out_v, [fj >> 7, fj & 127], vals)
                return carry2

            lax.fori_loop(0, _DPC, tile_body, 0)
            off = pl.multiple_of((row0 + c * chunk_rows) // 8, chunk_rows // 8)
            pltpu.sync_copy(out_v, out_hbm.at[pl.ds(off, chunk_rows // 8)])
            return carry

        lax.fori_loop(0, n_chunks, chunk_body, 0)

    return gather_k(table_grp, gidx3, sidx3)


def _tc_mlp(x, w1, b1, wp, bp, bt):
    b, d_in = x.shape
    h = w1.shape[1]

    def mlp_k(x_ref, w1_ref, b1_ref, wp_ref, bp_ref, o_ref):
        acc = jnp.dot(x_ref[...], w1_ref[...], preferred_element_type=jnp.float32)
        acc = jnp.maximum(acc + b1_ref[...], 0.0)
        out = jnp.dot(acc, wp_ref[...], preferred_element_type=jnp.float32)
        o_ref[...] = jax.nn.sigmoid(out + bp_ref[...])

    return pl.pallas_call(
        mlp_k,
        grid=(b // bt,),
        in_specs=[
            pl.BlockSpec((bt, d_in), lambda i: (i, 0)),
            pl.BlockSpec((d_in, h), lambda i: (0, 0)),
            pl.BlockSpec((1, h), lambda i: (0, 0)),
            pl.BlockSpec((h, 1), lambda i: (0, 0)),
            pl.BlockSpec((1, 1), lambda i: (0, 0)),
        ],
        out_specs=pl.BlockSpec((bt, 1), lambda i: (i, 0)),
        out_shape=jax.ShapeDtypeStruct((b, 1), jnp.float32),
    )(x, w1, b1, wp, bp)


def kernel(x_categorical, tables, W1, b1, Wp, bp):
    f, v, d = tables.shape
    b = x_categorical.shape[0]
    h = W1.shape[1]
    n_rows = b * f
    flat_idx = x_categorical + (jnp.arange(f, dtype=jnp.int32) * v)[None, :]
    gidx3 = ((flat_idx >> 6) * 8 + (flat_idx & 7)).reshape(
        _NW, (n_rows // _NW) // _IPD, _IPD)
    sidx3 = flat_idx.reshape(_NW, (n_rows // _NW) // _IPD, _IPD)
    table_grp = (tables.reshape(f * v // 64, 8, 8, d)
                 .swapaxes(1, 2).reshape(f * v // 8, 8 * d))
    emb8 = _sc_gather_grp(table_grp, gidx3, sidx3, n_rows)
    x = emb8.reshape(b, f * d)
    return _tc_mlp(x, W1, b1.reshape(1, h), Wp, bp.reshape(1, 1), 2048)


# speedup vs baseline: 1.1286x; 1.1286x over previous
import functools

import jax
import jax.numpy as jnp
from jax import lax
from jax.experimental import pallas as pl
from jax.experimental.pallas import tpu as pltpu
from jax.experimental.pallas import tpu_sc as plsc

_NC = 2
_NS = 16
_NW = _NC * _NS
_IPD = 128
_DPC = 8


def _sc_gather(table_flat, idx3, n_rows, d):
    rpw = n_rows // _NW
    n_dma = rpw // _IPD
    chunk_rows = _IPD * _DPC
    n_chunks = n_dma // _DPC
    mesh = plsc.VectorSubcoreMesh(core_axis_name="c", subcore_axis_name="s")

    @functools.partial(
        pl.kernel,
        out_type=jax.ShapeDtypeStruct((n_rows, d), jnp.float32),
        mesh=mesh,
        compiler_params=pltpu.CompilerParams(use_tc_tiling_on_sc=False),
        scratch_types=[
            pltpu.VMEM((n_dma, _IPD), jnp.int32),
            pltpu.VMEM((chunk_rows, d), jnp.float32),
            pltpu.SemaphoreType.DMA,
        ],
    )
    def gather_k(table_hbm, idx_hbm, out_hbm, idx_v, rows_v, gsem):
        wid = lax.axis_index("s") * _NC + lax.axis_index("c")
        row0 = wid * rpw
        pltpu.sync_copy(idx_hbm.at[wid], idx_v)

        def chunk_body(c, carry):
            copies = []
            for m in range(_DPC):
                cp = pltpu.async_copy(
                    table_hbm.at[idx_v.at[c * _DPC + m]],
                    rows_v.at[pl.ds(m * _IPD, _IPD)],
                    gsem,
                )
                copies.append(cp)
            for cp in copies:
                cp.wait()
            off = pl.multiple_of(row0 + c * chunk_rows, chunk_rows)
            pltpu.sync_copy(rows_v, out_hbm.at[pl.ds(off, chunk_rows)])
            return carry

        lax.fori_loop(0, n_chunks, chunk_body, 0)

    return gather_k(table_flat, idx3)


def _tc_mlp(x, w1, b1, wp, bp, bt):
    b, d_in = x.shape
    h = w1.shape[1]

    def mlp_k(x_ref, w1_ref, b1_ref, wp_ref, bp_ref, o_ref):
        acc = jnp.dot(x_ref[...], w1_ref[...], preferred_element_type=jnp.float32)
        acc = jnp.maximum(acc + b1_ref[...], 0.0)
        out = jnp.dot(acc, wp_ref[...], preferred_element_type=jnp.float32)
        o_ref[...] = jax.nn.sigmoid(out + bp_ref[...])

    return pl.pallas_call(
        mlp_k,
        grid=(b // bt,),
        in_specs=[
            pl.BlockSpec((bt, d_in), lambda i: (i, 0)),
            pl.BlockSpec((d_in, h), lambda i: (0, 0)),
            pl.BlockSpec((1, h), lambda i: (0, 0)),
            pl.BlockSpec((h, 1), lambda i: (0, 0)),
            pl.BlockSpec((1, 1), lambda i: (0, 0)),
        ],
        out_specs=pl.BlockSpec((bt, 1), lambda i: (i, 0)),
        out_shape=jax.ShapeDtypeStruct((b, 1), jnp.float32),
    )(x, w1, b1, wp, bp)


def kernel(x_categorical, tables, W1, b1, Wp, bp):
    f, v, d = tables.shape
    b = x_categorical.shape[0]
    h = W1.shape[1]
    n_rows = b * f
    flat_idx = x_categorical + (jnp.arange(f, dtype=jnp.int32) * v)[None, :]
    pidx = ((flat_idx & ~jnp.int32(63))
            | ((flat_idx & 7) << 3) | ((flat_idx >> 3) & 7))
    idx3 = pidx.reshape(_NW, (n_rows // _NW) // _IPD, _IPD)
    table_flat = (tables.reshape(f * v // 64, 8, 8, d)
                  .swapaxes(1, 2).reshape(f * v, d))
    emb = _sc_gather(table_flat, idx3, n_rows, d)
    x = emb.reshape(b, f * d)
    return _tc_mlp(x, W1, b1.reshape(1, h), Wp, bp.reshape(1, 1), 2048)


# speedup vs baseline: 2.9350x; 2.6006x over previous
import functools

import jax
import jax.numpy as jnp
from jax import lax
from jax.experimental import pallas as pl
from jax.experimental.pallas import tpu as pltpu
from jax.experimental.pallas import tpu_sc as plsc

_NC = 2
_NS = 16
_NW = _NC * _NS
_IDXS_PER_DMA = 128
_DMAS_PER_CHUNK = 8


def _sc_linearize(table_t, tail2, f, v, d):
    cw = 3840
    wn = 26
    mesh = plsc.VectorSubcoreMesh(core_axis_name="c", subcore_axis_name="s")

    @functools.partial(
        pl.kernel,
        out_type=jax.ShapeDtypeStruct((f * v * d,), jnp.float32),
        mesh=mesh,
        compiler_params=pltpu.CompilerParams(
            use_tc_tiling_on_sc=True, needs_layout_passes=False),
        scratch_types=[
            pltpu.VMEM((8, cw), jnp.float32),
            pltpu.VMEM((cw * d,), jnp.float32),
            pltpu.VMEM((104, 128), jnp.float32),
            pltpu.SemaphoreType.DMA,
        ],
    )
    def lin_k(t_hbm, tail_hbm, o_hbm, slab, outv, tailv, sem):
        wid = lax.axis_index("s") * _NC + lax.axis_index("c")
        lanes = lax.iota(jnp.int32, 16)

        def do_task(fi, col0, cwi):
            for half in range(2):
                pltpu.sync_copy(
                    t_hbm.at[pl.ds(fi * 16 + half * 8, 8), pl.ds(col0, cwi)],
                    slab.at[:, pl.ds(0, cwi)])

                def gbody(g, carry3):
                    tgt = g * 256 + lanes * 16 + half * 8
                    src = g * 16 + lanes
                    for dd in range(8):
                        vals = plsc.load_gather(
                            slab, [jnp.full((16,), dd, jnp.int32), src])
                        plsc.store_scatter(outv, [tgt + dd], vals)
                    return carry3

                lax.fori_loop(0, cwi // 16, gbody, 0)
            base = (fi * v + col0) * d
            pltpu.sync_copy(outv.at[pl.ds(0, cwi * d)],
                            o_hbm.at[pl.ds(base, cwi * d)])

        def fbody(fi, carry):
            def wbody(w, carry2):
                t = fi * (wn + 1) + w

                @pl.when((t & 31) == wid)
                def _():
                    do_task(fi, w * cw, cw)

                return carry2

            lax.fori_loop(0, wn, wbody, carry)
            t = fi * (wn + 1) + wn

            @pl.when((t & 31) == wid)
            def _():
                do_task(fi, wn * cw, 128)

            return carry

        lax.fori_loop(0, f, fbody, 0)

        @pl.when(wid == 31)
        def _():
            pltpu.sync_copy(tail_hbm, tailv)

            def tbody(fi, carry):
                for jg in range(2):
                    jj = jg * 16 + lanes
                    for dd in range(d):
                        sflat = (fi * 16 + dd) * 32 + jj
                        vals = plsc.load_gather(
                            tailv, [sflat >> 7, sflat & 127])
                        plsc.store_scatter(outv, [jj * 16 + dd], vals)
                base = (fi * v + (v - 32)) * d
                pltpu.sync_copy(outv.at[pl.ds(0, 32 * d)],
                                o_hbm.at[pl.ds(base, 32 * d)])
                return carry

            lax.fori_loop(0, f, tbody, 0)

    return lin_k(table_t, tail2)


def _sc_gather(table_flat, idx3, n_rows, d):
    rpw = n_rows // _NW
    n_dma = rpw // _IDXS_PER_DMA
    chunk_rows = _IDXS_PER_DMA * _DMAS_PER_CHUNK
    n_chunks = n_dma // _DMAS_PER_CHUNK
    mesh = plsc.VectorSubcoreMesh(core_axis_name="c", subcore_axis_name="s")

    @functools.partial(
        pl.kernel,
        out_type=jax.ShapeDtypeStruct((n_rows, d), jnp.float32),
        mesh=mesh,
        compiler_params=pltpu.CompilerParams(use_tc_tiling_on_sc=False),
        scratch_types=[
            pltpu.VMEM((n_dma, _IDXS_PER_DMA), jnp.int32),
            pltpu.VMEM((chunk_rows, d), jnp.float32),
            pltpu.SemaphoreType.DMA,
        ],
    )
    def gather_k(table_hbm, idx_hbm, out_hbm, idx_v, rows_v, gsem):
        wid = lax.axis_index("s") * _NC + lax.axis_index("c")
        row0 = wid * rpw
        pltpu.sync_copy(idx_hbm.at[wid], idx_v)

        def chunk_body(c, carry):
            copies = []
            for m in range(_DMAS_PER_CHUNK):
                cp = pltpu.async_copy(
                    table_hbm.at[idx_v.at[c * _DMAS_PER_CHUNK + m]],
                    rows_v.at[pl.ds(m * _IDXS_PER_DMA, _IDXS_PER_DMA)],
                    gsem,
                )
                copies.append(cp)
            for cp in copies:
                cp.wait()
            off = pl.multiple_of(row0 + c * chunk_rows, chunk_rows)
            pltpu.sync_copy(rows_v, out_hbm.at[pl.ds(off, chunk_rows)])
            return carry

        lax.fori_loop(0, n_chunks, chunk_body, 0)

    return gather_k(table_flat, idx3)


def _tc_mlp(x, w1, b1, wp, bp, bt):
    b, d_in = x.shape
    h = w1.shape[1]

    def mlp_k(x_ref, w1_ref, b1_ref, wp_ref, bp_ref, o_ref):
        acc = jnp.dot(x_ref[...], w1_ref[...], preferred_element_type=jnp.float32)
        acc = jnp.maximum(acc + b1_ref[...], 0.0)
        out = jnp.dot(acc, wp_ref[...], preferred_element_type=jnp.float32)
        o_ref[...] = jax.nn.sigmoid(out + bp_ref[...])

    return pl.pallas_call(
        mlp_k,
        grid=(b // bt,),
        in_specs=[
            pl.BlockSpec((bt, d_in), lambda i: (i, 0)),
            pl.BlockSpec((d_in, h), lambda i: (0, 0)),
            pl.BlockSpec((1, h), lambda i: (0, 0)),
            pl.BlockSpec((h, 1), lambda i: (0, 0)),
            pl.BlockSpec((1, 1), lambda i: (0, 0)),
        ],
        out_specs=pl.BlockSpec((bt, 1), lambda i: (i, 0)),
        out_shape=jax.ShapeDtypeStruct((b, 1), jnp.float32),
    )(x, w1, b1, wp, bp)


def kernel(x_categorical, tables, W1, b1, Wp, bp):
    f, v, d = tables.shape
    b = x_categorical.shape[0]
    h = W1.shape[1]
    n_rows = b * f
    flat_idx = x_categorical + (jnp.arange(f, dtype=jnp.int32) * v)[None, :]
    idx3 = flat_idx.reshape(_NW, (n_rows // _NW) // _IDXS_PER_DMA, _IDXS_PER_DMA)
    table_t = tables.transpose(0, 2, 1).reshape(f * d, v)
    tail2 = (tables[:, v - 32:, :].transpose(0, 2, 1)
             .reshape(f * d * 32 // 128, 128))
    table_flat = _sc_linearize(table_t, tail2, f, v, d).reshape(f * v, d)
    emb = _sc_gather(table_flat, idx3, n_rows, d)
    x = emb.reshape(b, f * d)
    return _tc_mlp(x, W1, b1.reshape(1, h), Wp, bp.reshape(1, 1), 2048)


# speedup vs baseline: 3.2075x; 1.0928x over previous
import functools

import jax
import jax.numpy as jnp
from jax import lax
from jax.experimental import pallas as pl
from jax.experimental.pallas import tpu as pltpu
from jax.experimental.pallas import tpu_sc as plsc

_NC = 2
_NS = 16
_NW = _NC * _NS
_IDXS_PER_DMA = 128
_DMAS_PER_CHUNK = 8


def _sc_linearize(table_t, tail2, f, v, d):
    cw = 3328
    wn = 30
    mesh = plsc.VectorSubcoreMesh(core_axis_name="c", subcore_axis_name="s")

    @functools.partial(
        pl.kernel,
        out_type=jax.ShapeDtypeStruct((f * v * d,), jnp.float32),
        mesh=mesh,
        compiler_params=pltpu.CompilerParams(
            use_tc_tiling_on_sc=True, needs_layout_passes=False),
        scratch_types=[
            pltpu.VMEM((16, cw), jnp.float32),
            pltpu.VMEM((cw * d,), jnp.float32),
            pltpu.VMEM((104, 128), jnp.float32),
            pltpu.SemaphoreType.DMA,
        ],
    )
    def lin_k(t_hbm, tail_hbm, o_hbm, slab, outv, tailv, sem):
        wid = lax.axis_index("s") * _NC + lax.axis_index("c")
        lanes = lax.iota(jnp.int32, 16)

        def do_task(fi, col0, cwi):
            cps = []
            for half in range(2):
                cps.append(pltpu.async_copy(
                    t_hbm.at[pl.ds(fi * 16 + half * 8, 8), pl.ds(col0, cwi)],
                    slab.at[pl.ds(half * 8, 8), pl.ds(0, cwi)], sem))
            for cp in cps:
                cp.wait()

            def gbody(g, carry3):
                tgt = g * 256 + lanes * 16
                for dd in range(16):
                    vals = slab[dd, pl.ds(g * 16, 16)]
                    plsc.store_scatter(outv, [tgt + dd], vals)
                return carry3

            lax.fori_loop(0, cwi // 16, gbody, 0, unroll=2)
            base = (fi * v + col0) * d
            pltpu.sync_copy(outv.at[pl.ds(0, cwi * d)],
                            o_hbm.at[pl.ds(base, cwi * d)])

        def fbody(fi, carry):
            def wbody(w, carry2):
                t = fi * (wn + 1) + w

                @pl.when((t & 31) == wid)
                def _():
                    do_task(fi, w * cw, cw)

                return carry2

            lax.fori_loop(0, wn, wbody, carry)
            t = fi * (wn + 1) + wn

            @pl.when((t & 31) == wid)
            def _():
                do_task(fi, wn * cw, 128)

            return carry

        lax.fori_loop(0, f, fbody, 0)

        @pl.when(wid == 31)
        def _():
            pltpu.sync_copy(tail_hbm, tailv)

            def tbody(fi, carry):
                for jg in range(2):
                    jj = jg * 16 + lanes
                    for dd in range(d):
                        sflat = (fi * 16 + dd) * 32 + jj
                        vals = plsc.load_gather(
                            tailv, [sflat >> 7, sflat & 127])
                        plsc.store_scatter(outv, [jj * 16 + dd], vals)
                base = (fi * v + (v - 32)) * d
                pltpu.sync_copy(outv.at[pl.ds(0, 32 * d)],
                                o_hbm.at[pl.ds(base, 32 * d)])
                return carry

            lax.fori_loop(0, f, tbody, 0)

    return lin_k(table_t, tail2)


def _sc_gather(table_flat, idx3, n_rows, d):
    rpw = n_rows // _NW
    n_dma = rpw // _IDXS_PER_DMA
    chunk_rows = _IDXS_PER_DMA * _DMAS_PER_CHUNK
    n_chunks = n_dma // _DMAS_PER_CHUNK
    mesh = plsc.VectorSubcoreMesh(core_axis_name="c", subcore_axis_name="s")

    @functools.partial(
        pl.kernel,
        out_type=jax.ShapeDtypeStruct((n_rows, d), jnp.float32),
        mesh=mesh,
        compiler_params=pltpu.CompilerParams(use_tc_tiling_on_sc=False),
        scratch_types=[
            pltpu.VMEM((n_dma, _IDXS_PER_DMA), jnp.int32),
            pltpu.VMEM((chunk_rows, d), jnp.float32),
            pltpu.SemaphoreType.DMA,
        ],
    )
    def gather_k(table_hbm, idx_hbm, out_hbm, idx_v, rows_v, gsem):
        wid = lax.axis_index("s") * _NC + lax.axis_index("c")
        row0 = wid * rpw
        pltpu.sync_copy(idx_hbm.at[wid], idx_v)

        def chunk_body(c, carry):
            copies = []
            for m in range(_DMAS_PER_CHUNK):
                cp = pltpu.async_copy(
                    table_hbm.at[idx_v.at[c * _DMAS_PER_CHUNK + m]],
                    rows_v.at[pl.ds(m * _IDXS_PER_DMA, _IDXS_PER_DMA)],
                    gsem,
                )
                copies.append(cp)
            for cp in copies:
                cp.wait()
            off = pl.multiple_of(row0 + c * chunk_rows, chunk_rows)
            pltpu.sync_copy(rows_v, out_hbm.at[pl.ds(off, chunk_rows)])
            return carry

        lax.fori_loop(0, n_chunks, chunk_body, 0)

    return gather_k(table_flat, idx3)


def _tc_mlp(x, w1, b1, wp, bp, bt):
    b, d_in = x.shape
    h = w1.shape[1]

    def mlp_k(x_ref, w1_ref, b1_ref, wp_ref, bp_ref, o_ref):
        acc = jnp.dot(x_ref[...], w1_ref[...], preferred_element_type=jnp.float32)
        acc = jnp.maximum(acc + b1_ref[...], 0.0)
        out = jnp.dot(acc, wp_ref[...], preferred_element_type=jnp.float32)
        o_ref[...] = jax.nn.sigmoid(out + bp_ref[...])

    return pl.pallas_call(
        mlp_k,
        grid=(b // bt,),
        in_specs=[
            pl.BlockSpec((bt, d_in), lambda i: (i, 0)),
            pl.BlockSpec((d_in, h), lambda i: (0, 0)),
            pl.BlockSpec((1, h), lambda i: (0, 0)),
            pl.BlockSpec((h, 1), lambda i: (0, 0)),
            pl.BlockSpec((1, 1), lambda i: (0, 0)),
        ],
        out_specs=pl.BlockSpec((bt, 1), lambda i: (i, 0)),
        out_shape=jax.ShapeDtypeStruct((b, 1), jnp.float32),
    )(x, w1, b1, wp, bp)


def kernel(x_categorical, tables, W1, b1, Wp, bp):
    f, v, d = tables.shape
    b = x_categorical.shape[0]
    h = W1.shape[1]
    n_rows = b * f
    flat_idx = x_categorical + (jnp.arange(f, dtype=jnp.int32) * v)[None, :]
    idx3 = flat_idx.reshape(_NW, (n_rows // _NW) // _IDXS_PER_DMA, _IDXS_PER_DMA)
    table_t = tables.transpose(0, 2, 1).reshape(f * d, v)
    tail2 = (tables[:, v - 32:, :].transpose(0, 2, 1)
             .reshape(f * d * 32 // 128, 128))
    table_flat = _sc_linearize(table_t, tail2, f, v, d).reshape(f * v, d)
    emb = _sc_gather(table_flat, idx3, n_rows, d)
    x = emb.reshape(b, f * d)
    return _tc_mlp(x, W1, b1.reshape(1, h), Wp, bp.reshape(1, 1), 2048)


# speedup vs baseline: 3.2091x; 1.0005x over previous
import functools

import jax
import jax.numpy as jnp
from jax import lax
from jax.experimental import pallas as pl
from jax.experimental.pallas import tpu as pltpu
from jax.experimental.pallas import tpu_sc as plsc

_NC = 2
_NS = 16
_NW = _NC * _NS
_IDXS_PER_DMA = 128
_DMAS_PER_CHUNK = 8


def _sc_linearize(table_t, tail2, f, v, d):
    cw = 3328
    wn = 30
    mesh = plsc.VectorSubcoreMesh(core_axis_name="c", subcore_axis_name="s")

    @functools.partial(
        pl.kernel,
        out_type=jax.ShapeDtypeStruct((f * v * d,), jnp.float32),
        mesh=mesh,
        compiler_params=pltpu.CompilerParams(
            use_tc_tiling_on_sc=True, needs_layout_passes=False),
        scratch_types=[
            pltpu.VMEM((16, cw), jnp.float32),
            pltpu.VMEM((cw * d,), jnp.float32),
            pltpu.VMEM((104, 128), jnp.float32),
            pltpu.SemaphoreType.DMA,
        ],
    )
    def lin_k(t_hbm, tail_hbm, o_hbm, slab, outv, tailv, sem):
        wid = lax.axis_index("s") * _NC + lax.axis_index("c")
        lanes = lax.iota(jnp.int32, 16)

        def do_task(fi, col0, cwi):
            cps = []
            for half in range(2):
                cps.append(pltpu.async_copy(
                    t_hbm.at[pl.ds(fi * 16 + half * 8, 8), pl.ds(col0, cwi)],
                    slab.at[pl.ds(half * 8, 8), pl.ds(0, cwi)], sem))
            for cp in cps:
                cp.wait()

            def gbody(g, carry3):
                tgt = g * 256 + lanes * 16
                for dd in range(16):
                    vals = slab[dd, pl.ds(g * 16, 16)]
                    plsc.store_scatter(outv, [tgt + dd], vals)
                return carry3

            lax.fori_loop(0, cwi // 16, gbody, 0, unroll=4)
            base = (fi * v + col0) * d
            pltpu.sync_copy(outv.at[pl.ds(0, cwi * d)],
                            o_hbm.at[pl.ds(base, cwi * d)])

        def fbody(fi, carry):
            def wbody(w, carry2):
                t = fi * (wn + 1) + w

                @pl.when((t & 31) == wid)
                def _():
                    do_task(fi, w * cw, cw)

                return carry2

            lax.fori_loop(0, wn, wbody, carry)
            t = fi * (wn + 1) + wn

            @pl.when((t & 31) == wid)
            def _():
                do_task(fi, wn * cw, 128)

            return carry

        lax.fori_loop(0, f, fbody, 0)

        @pl.when(wid == 31)
        def _():
            pltpu.sync_copy(tail_hbm, tailv)

            def tbody(fi, carry):
                for jg in range(2):
                    jj = jg * 16 + lanes
                    for dd in range(d):
                        sflat = (fi * 16 + dd) * 32 + jj
                        vals = plsc.load_gather(
                            tailv, [sflat >> 7, sflat & 127])
                        plsc.store_scatter(outv, [jj * 16 + dd], vals)
                base = (fi * v + (v - 32)) * d
                pltpu.sync_copy(outv.at[pl.ds(0, 32 * d)],
                                o_hbm.at[pl.ds(base, 32 * d)])
                return carry

            lax.fori_loop(0, f, tbody, 0)

    return lin_k(table_t, tail2)


def _sc_gather(table_flat, idx3, n_rows, d):
    rpw = n_rows // _NW
    n_dma = rpw // _IDXS_PER_DMA
    chunk_rows = _IDXS_PER_DMA * _DMAS_PER_CHUNK
    n_chunks = n_dma // _DMAS_PER_CHUNK
    mesh = plsc.VectorSubcoreMesh(core_axis_name="c", subcore_axis_name="s")

    @functools.partial(
        pl.kernel,
        out_type=jax.ShapeDtypeStruct((n_rows, d), jnp.float32),
        mesh=mesh,
        compiler_params=pltpu.CompilerParams(use_tc_tiling_on_sc=False),
        scratch_types=[
            pltpu.VMEM((n_dma, _IDXS_PER_DMA), jnp.int32),
            pltpu.VMEM((chunk_rows, d), jnp.float32),
            pltpu.SemaphoreType.DMA,
        ],
    )
    def gather_k(table_hbm, idx_hbm, out_hbm, idx_v, rows_v, gsem):
        wid = lax.axis_index("s") * _NC + lax.axis_index("c")
        row0 = wid * rpw
        pltpu.sync_copy(idx_hbm.at[wid], idx_v)

        def chunk_body(c, carry):
            copies = []
            for m in range(_DMAS_PER_CHUNK):
                cp = pltpu.async_copy(
                    table_hbm.at[idx_v.at[c * _DMAS_PER_CHUNK + m]],
                    rows_v.at[pl.ds(m * _IDXS_PER_DMA, _IDXS_PER_DMA)],
                    gsem,
                )
                copies.append(cp)
            for cp in copies:
                cp.wait()
            off = pl.multiple_of(row0 + c * chunk_rows, chunk_rows)
            pltpu.sync_copy(rows_v, out_hbm.at[pl.ds(off, chunk_rows)])
            return carry

        lax.fori_loop(0, n_chunks, chunk_body, 0)

    return gather_k(table_flat, idx3)


def _tc_mlp(x, w1, b1, wp, bp, bt):
    b, d_in = x.shape
    h = w1.shape[1]

    def mlp_k(x_ref, w1_ref, b1_ref, wp_ref, bp_ref, o_ref):
        acc = jnp.dot(x_ref[...], w1_ref[...], preferred_element_type=jnp.float32)
        acc = jnp.maximum(acc + b1_ref[...], 0.0)
        out = jnp.dot(acc, wp_ref[...], preferred_element_type=jnp.float32)
        o_ref[...] = jax.nn.sigmoid(out + bp_ref[...])

    return pl.pallas_call(
        mlp_k,
        grid=(b // bt,),
        in_specs=[
            pl.BlockSpec((bt, d_in), lambda i: (i, 0)),
            pl.BlockSpec((d_in, h), lambda i: (0, 0)),
            pl.BlockSpec((1, h), lambda i: (0, 0)),
            pl.BlockSpec((h, 1), lambda i: (0, 0)),
            pl.BlockSpec((1, 1), lambda i: (0, 0)),
        ],
        out_specs=pl.BlockSpec((bt, 1), lambda i: (i, 0)),
        out_shape=jax.ShapeDtypeStruct((b, 1), jnp.float32),
    )(x, w1, b1, wp, bp)


def kernel(x_categorical, tables, W1, b1, Wp, bp):
    f, v, d = tables.shape
    b = x_categorical.shape[0]
    h = W1.shape[1]
    n_rows = b * f
    flat_idx = x_categorical + (jnp.arange(f, dtype=jnp.int32) * v)[None, :]
    idx3 = flat_idx.reshape(_NW, (n_rows // _NW) // _IDXS_PER_DMA, _IDXS_PER_DMA)
    table_t = tables.transpose(0, 2, 1).reshape(f * d, v)
    tail2 = (tables[:, v - 32:, :].transpose(0, 2, 1)
             .reshape(f * d * 32 // 128, 128))
    table_flat = _sc_linearize(table_t, tail2, f, v, d).reshape(f * v, d)
    emb = _sc_gather(table_flat, idx3, n_rows, d)
    x = emb.reshape(b, f * d)
    return _tc_mlp(x, W1, b1.reshape(1, h), Wp, bp.reshape(1, 1), 2048)


# speedup vs baseline: 3.3312x; 1.0381x over previous
import functools

import jax
import jax.numpy as jnp
from jax import lax
from jax.experimental import pallas as pl
from jax.experimental.pallas import tpu as pltpu
from jax.experimental.pallas import tpu_sc as plsc

_NC = 2
_NS = 16
_NW = _NC * _NS
_IDXS_PER_DMA = 128
_DMAS_PER_CHUNK = 8


def _sc_linearize(table_t, tail2, f, v, d):
    cw = 3328
    wn = 30
    mesh = plsc.VectorSubcoreMesh(core_axis_name="c", subcore_axis_name="s")

    @functools.partial(
        pl.kernel,
        out_type=jax.ShapeDtypeStruct((f * v * d,), jnp.float32),
        mesh=mesh,
        compiler_params=pltpu.CompilerParams(
            use_tc_tiling_on_sc=True, needs_layout_passes=False),
        scratch_types=[
            pltpu.VMEM((16, cw), jnp.float32),
            pltpu.VMEM((cw * d,), jnp.float32),
            pltpu.VMEM((104, 128), jnp.float32),
            pltpu.SemaphoreType.DMA,
        ],
    )
    def lin_k(t_hbm, tail_hbm, o_hbm, slab, outv, tailv, sem):
        wid = lax.axis_index("s") * _NC + lax.axis_index("c")
        lanes = lax.iota(jnp.int32, 16)

        def do_task(fi, col0, cwi):
            cps = []
            for half in range(2):
                cps.append(pltpu.async_copy(
                    t_hbm.at[pl.ds(fi * 16 + half * 8, 8), pl.ds(col0, cwi)],
                    slab.at[pl.ds(half * 8, 8), pl.ds(0, cwi)], sem))
            for half in range(2):
                cps[half].wait()

                def gbody(g, carry3, h=half):
                    tgt = g * 256 + lanes * 16 + h * 8
                    for dd in range(8):
                        vals = slab[h * 8 + dd, pl.ds(g * 16, 16)]
                        plsc.store_scatter(outv, [tgt + dd], vals)
                    return carry3

                lax.fori_loop(0, cwi // 16, gbody, 0, unroll=4)
            base = (fi * v + col0) * d
            pltpu.sync_copy(outv.at[pl.ds(0, cwi * d)],
                            o_hbm.at[pl.ds(base, cwi * d)])

        def fbody(fi, carry):
            def wbody(w, carry2):
                t = fi * (wn + 1) + w

                @pl.when((t & 31) == wid)
                def _():
                    do_task(fi, w * cw, cw)

                return carry2

            lax.fori_loop(0, wn, wbody, carry)
            t = fi * (wn + 1) + wn

            @pl.when((t & 31) == wid)
            def _():
                do_task(fi, wn * cw, 128)

            return carry

        lax.fori_loop(0, f, fbody, 0)

        @pl.when(wid == 31)
        def _():
            pltpu.sync_copy(tail_hbm, tailv)

            def tbody(fi, carry):
                for jg in range(2):
                    jj = jg * 16 + lanes
                    for dd in range(d):
                        sflat = (fi * 16 + dd) * 32 + jj
                        vals = plsc.load_gather(
                            tailv, [sflat >> 7, sflat & 127])
                        plsc.store_scatter(outv, [jj * 16 + dd], vals)
                base = (fi * v + (v - 32)) * d
                pltpu.sync_copy(outv.at[pl.ds(0, 32 * d)],
                                o_hbm.at[pl.ds(base, 32 * d)])
                return carry

            lax.fori_loop(0, f, tbody, 0)

    return lin_k(table_t, tail2)


def _sc_gather(table_flat, idx3, n_rows, d):
    rpw = n_rows // _NW
    n_dma = rpw // _IDXS_PER_DMA
    chunk_rows = _IDXS_PER_DMA * _DMAS_PER_CHUNK
    n_chunks = n_dma // _DMAS_PER_CHUNK
    mesh = plsc.VectorSubcoreMesh(core_axis_name="c", subcore_axis_name="s")

    @functools.partial(
        pl.kernel,
        out_type=jax.ShapeDtypeStruct((n_rows, d), jnp.float32),
        mesh=mesh,
        compiler_params=pltpu.CompilerParams(use_tc_tiling_on_sc=False),
        scratch_types=[
            pltpu.VMEM((n_dma, _IDXS_PER_DMA), jnp.int32),
            pltpu.VMEM((chunk_rows, d), jnp.float32),
            pltpu.SemaphoreType.DMA,
        ],
    )
    def gather_k(table_hbm, idx_hbm, out_hbm, idx_v, rows_v, gsem):
        wid = lax.axis_index("s") * _NC + lax.axis_index("c")
        row0 = wid * rpw
        pltpu.sync_copy(idx_hbm.at[wid], idx_v)

        def chunk_body(c, carry):
            copies = []
            for m in range(_DMAS_PER_CHUNK):
                cp = pltpu.async_copy(
                    table_hbm.at[idx_v.at[c * _DMAS_PER_CHUNK + m]],
                    rows_v.at[pl.ds(m * _IDXS_PER_DMA, _IDXS_PER_DMA)],
                    gsem,
                )
                copies.append(cp)
            for cp in copies:
                cp.wait()
            off = pl.multiple_of(row0 + c * chunk_rows, chunk_rows)
            pltpu.sync_copy(rows_v, out_hbm.at[pl.ds(off, chunk_rows)])
            return carry

        lax.fori_loop(0, n_chunks, chunk_body, 0)

    return gather_k(table_flat, idx3)


def _tc_mlp(x, w1, b1, wp, bp, bt):
    b, d_in = x.shape
    h = w1.shape[1]

    def mlp_k(x_ref, w1_ref, b1_ref, wp_ref, bp_ref, o_ref):
        acc = jnp.dot(x_ref[...], w1_ref[...], preferred_element_type=jnp.float32)
        acc = jnp.maximum(acc + b1_ref[...], 0.0)
        out = jnp.dot(acc, wp_ref[...], preferred_element_type=jnp.float32)
        o_ref[...] = jax.nn.sigmoid(out + bp_ref[...])

    return pl.pallas_call(
        mlp_k,
        grid=(b // bt,),
        in_specs=[
            pl.BlockSpec((bt, d_in), lambda i: (i, 0)),
            pl.BlockSpec((d_in, h), lambda i: (0, 0)),
            pl.BlockSpec((1, h), lambda i: (0, 0)),
            pl.BlockSpec((h, 1), lambda i: (0, 0)),
            pl.BlockSpec((1, 1), lambda i: (0, 0)),
        ],
        out_specs=pl.BlockSpec((bt, 1), lambda i: (i, 0)),
        out_shape=jax.ShapeDtypeStruct((b, 1), jnp.float32),
    )(x, w1, b1, wp, bp)


def kernel(x_categorical, tables, W1, b1, Wp, bp):
    f, v, d = tables.shape
    b = x_categorical.shape[0]
    h = W1.shape[1]
    n_rows = b * f
    flat_idx = x_categorical + (jnp.arange(f, dtype=jnp.int32) * v)[None, :]
    idx3 = flat_idx.reshape(_NW, (n_rows // _NW) // _IDXS_PER_DMA, _IDXS_PER_DMA)
    table_t = tables.transpose(0, 2, 1).reshape(f * d, v)
    tail2 = (tables[:, v - 32:, :].transpose(0, 2, 1)
             .reshape(f * d * 32 // 128, 128))
    table_flat = _sc_linearize(table_t, tail2, f, v, d).reshape(f * v, d)
    emb = _sc_gather(table_flat, idx3, n_rows, d)
    x = emb.reshape(b, f * d)
    return _tc_mlp(x, W1, b1.reshape(1, h), Wp, bp.reshape(1, 1), 2048)
